# baseline (speedup 1.0000x reference)
import functools

import jax
import jax.numpy as jnp
from jax import lax
from jax.experimental import pallas as pl
from jax.experimental.pallas import tpu as pltpu
from jax.experimental.pallas import tpu_sc as plsc

VOCAB = 100000
EMB_DIM = 128
UNITS = 512


_VPAD = 100352
_VBLK = 2048
_GRID = _VPAD // _VBLK


def _lut_body(tab_ref, w1_ref, b1_ref, w2_ref, b2_ref, out_ref):
    h = jnp.maximum(tab_ref[...], 0.0).astype(jnp.bfloat16)
    a = jnp.dot(h, w1_ref[...], preferred_element_type=jnp.float32) + b1_ref[...]
    a = jnp.maximum(a, 0.0)
    o = jnp.dot(a, w2_ref[...], preferred_element_type=jnp.float32) + b2_ref[...]
    out_ref[...] = o.reshape(_VBLK // 128, 128)


def _build_lut(table, W1, b1, W2, b2):
    b1r = b1.reshape(1, UNITS)
    b2r = b2.reshape(1, 1)
    out = pl.pallas_call(
        _lut_body,
        grid=(_GRID,),
        in_specs=[
            pl.BlockSpec((_VBLK, EMB_DIM), lambda i: (i, 0)),
            pl.BlockSpec((EMB_DIM, UNITS), lambda i: (0, 0)),
            pl.BlockSpec((1, UNITS), lambda i: (0, 0)),
            pl.BlockSpec((UNITS, 1), lambda i: (0, 0)),
            pl.BlockSpec((1, 1), lambda i: (0, 0)),
        ],
        out_specs=pl.BlockSpec((_VBLK // 128, 128), lambda i: (i, 0)),
        out_shape=jax.ShapeDtypeStruct((_VPAD // 128, 128), jnp.float32),
    )(table, W1.astype(jnp.bfloat16), b1r, W2, b2r)
    return out.reshape(_VPAD)


_NC = 2
_NS = 16
_NW = _NC * _NS


_CH = 256
_K = 16


def _gather_body(rows_per_w, n_fields, lut_hbm, idx_hbm, out_hbm,
                 lut_sp, idx_v, out_v, sem):
    sid = lax.axis_index("s")
    wid = sid * _NC + lax.axis_index("c")
    seg = _VPAD // _NS
    pltpu.sync_copy(lut_hbm.at[pl.ds(sid * seg, seg)],
                    lut_sp.at[pl.ds(sid * seg, seg)])
    plsc.subcore_barrier()
    rb = wid * rows_per_w

    def chunk_body(c, _):
        r0 = rb + c * _CH
        pltpu.sync_copy(idx_hbm.at[pl.ds(r0, _CH), :], idx_v)

        def grp_body(g, _):
            handles = [
                pltpu.async_copy(lut_sp.at[idx_v.at[g * _K + k]],
                                 out_v.at[g * _K + k], sem)
                for k in range(_K)
            ]
            for h in handles:
                h.wait()
            return 0

        lax.fori_loop(0, _CH // _K, grp_body, 0)
        pltpu.sync_copy(out_v, out_hbm.at[pl.ds(r0, _CH), :])
        return 0

    lax.fori_loop(0, rows_per_w // _CH, chunk_body, 0)


def _gather(lut, x):
    B, F = x.shape
    rows_per_w = B // _NW
    mesh = plsc.VectorSubcoreMesh(core_axis_name="c", subcore_axis_name="s")
    return pl.kernel(
        functools.partial(_gather_body, rows_per_w, F),
        out_type=jax.ShapeDtypeStruct((B, F), jnp.float32),
        mesh=mesh,
        scratch_types=[
            pltpu.VMEM_SHARED((_VPAD,), jnp.float32),
            pltpu.VMEM((_CH, F), jnp.int32),
            pltpu.VMEM((_CH, F), jnp.float32),
            pltpu.SemaphoreType.DMA,
        ],
    )(lut, x)


def kernel(x, table, W1, b1, W2, b2):
    lut = _build_lut(table, W1, b1, W2, b2)
    B, F = x.shape
    out2d = _gather(lut, x)
    return out2d.reshape(B, F, 1)


# speedup vs baseline: 1.0685x; 1.0685x over previous
import functools

import jax
import jax.numpy as jnp
from jax import lax
from jax.experimental import pallas as pl
from jax.experimental.pallas import tpu as pltpu
from jax.experimental.pallas import tpu_sc as plsc

VOCAB = 100000
EMB_DIM = 128
UNITS = 512


_VPAD = 100352
_VBLK = 7168
_GRID = _VPAD // _VBLK


def _lut_body(tab_ref, w1_ref, b1_ref, w2_ref, b2_ref, out_ref):
    h = jnp.maximum(tab_ref[...], 0.0).astype(jnp.bfloat16)
    a = jnp.dot(h, w1_ref[...], preferred_element_type=jnp.float32) + b1_ref[...]
    a = jnp.maximum(a, 0.0)
    o = jnp.dot(a, w2_ref[...], preferred_element_type=jnp.float32) + b2_ref[...]
    out_ref[...] = o.reshape(_VBLK // 128, 128)


def _build_lut(table, W1, b1, W2, b2):
    b1r = b1.reshape(1, UNITS)
    b2r = b2.reshape(1, 1)
    out = pl.pallas_call(
        _lut_body,
        grid=(_GRID,),
        in_specs=[
            pl.BlockSpec((_VBLK, EMB_DIM), lambda i: (i, 0)),
            pl.BlockSpec((EMB_DIM, UNITS), lambda i: (0, 0)),
            pl.BlockSpec((1, UNITS), lambda i: (0, 0)),
            pl.BlockSpec((UNITS, 1), lambda i: (0, 0)),
            pl.BlockSpec((1, 1), lambda i: (0, 0)),
        ],
        out_specs=pl.BlockSpec((_VBLK // 128, 128), lambda i: (i, 0)),
        out_shape=jax.ShapeDtypeStruct((_VPAD // 128, 128), jnp.float32),
    )(table, W1.astype(jnp.bfloat16), b1r, W2, b2r)
    return out.reshape(_VPAD)


_NC = 2
_NS = 16
_NW = _NC * _NS


_CH = 256
_K = 16


def _gather_body(rows_per_w, n_fields, lut_hbm, idx_hbm, out_hbm,
                 lut_sp, idx_v, out_v, sem):
    sid = lax.axis_index("s")
    wid = sid * _NC + lax.axis_index("c")
    seg = _VPAD // _NS
    pltpu.sync_copy(lut_hbm.at[pl.ds(sid * seg, seg)],
                    lut_sp.at[pl.ds(sid * seg, seg)])
    plsc.subcore_barrier()
    rb = wid * rows_per_w

    def chunk_body(c, _):
        r0 = rb + c * _CH
        pltpu.sync_copy(idx_hbm.at[pl.ds(r0, _CH), :], idx_v)

        def grp_body(g, _):
            handles = [
                pltpu.async_copy(lut_sp.at[idx_v.at[g * _K + k]],
                                 out_v.at[g * _K + k], sem)
                for k in range(_K)
            ]
            for h in handles:
                h.wait()
            return 0

        lax.fori_loop(0, _CH // _K, grp_body, 0)
        pltpu.sync_copy(out_v, out_hbm.at[pl.ds(r0, _CH), :])
        return 0

    lax.fori_loop(0, rows_per_w // _CH, chunk_body, 0)


def _gather(lut, x):
    B, F = x.shape
    rows_per_w = B // _NW
    mesh = plsc.VectorSubcoreMesh(core_axis_name="c", subcore_axis_name="s")
    return pl.kernel(
        functools.partial(_gather_body, rows_per_w, F),
        out_type=jax.ShapeDtypeStruct((B, F), jnp.float32),
        mesh=mesh,
        scratch_types=[
            pltpu.VMEM_SHARED((_VPAD,), jnp.float32),
            pltpu.VMEM((_CH, F), jnp.int32),
            pltpu.VMEM((_CH, F), jnp.float32),
            pltpu.SemaphoreType.DMA,
        ],
    )(lut, x)


def kernel(x, table, W1, b1, W2, b2):
    lut = _build_lut(table, W1, b1, W2, b2)
    B, F = x.shape
    out2d = _gather(lut, x)
    return out2d.reshape(B, F, 1)


# speedup vs baseline: 1.0790x; 1.0098x over previous
import functools

import jax
import jax.numpy as jnp
from jax import lax
from jax.experimental import pallas as pl
from jax.experimental.pallas import tpu as pltpu
from jax.experimental.pallas import tpu_sc as plsc

VOCAB = 100000
EMB_DIM = 128
UNITS = 512


_VPAD = 100352
_VBLK = 7168
_GRID = _VPAD // _VBLK


def _lut_body(tab_ref, w1_ref, b1_ref, w2_ref, b2_ref, out_ref):
    h = jnp.maximum(tab_ref[...], 0.0).astype(jnp.bfloat16)
    a = jnp.dot(h, w1_ref[...], preferred_element_type=jnp.float32) + b1_ref[...]
    a = jnp.maximum(a, 0.0)
    o = jnp.dot(a, w2_ref[...], preferred_element_type=jnp.float32) + b2_ref[...]
    out_ref[...] = o.reshape(_VBLK // 128, 128)


def _build_lut(table, W1, b1, W2, b2):
    b1r = b1.reshape(1, UNITS)
    b2r = b2.reshape(1, 1)
    out = pl.pallas_call(
        _lut_body,
        grid=(_GRID,),
        in_specs=[
            pl.BlockSpec((_VBLK, EMB_DIM), lambda i: (i, 0)),
            pl.BlockSpec((EMB_DIM, UNITS), lambda i: (0, 0)),
            pl.BlockSpec((1, UNITS), lambda i: (0, 0)),
            pl.BlockSpec((UNITS, 1), lambda i: (0, 0)),
            pl.BlockSpec((1, 1), lambda i: (0, 0)),
        ],
        out_specs=pl.BlockSpec((_VBLK // 128, 128), lambda i: (i, 0)),
        out_shape=jax.ShapeDtypeStruct((_VPAD // 128, 128), jnp.float32),
    )(table, W1.astype(jnp.bfloat16), b1r, W2, b2r)
    return out.reshape(_VPAD)


_NC = 2
_NS = 16
_NW = _NC * _NS


_CH = 256
_K = 32


def _gather_body(rows_per_w, n_fields, lut_hbm, idx_hbm, out_hbm,
                 lut_sp, idx_v, out_v, sem):
    sid = lax.axis_index("s")
    wid = sid * _NC + lax.axis_index("c")
    seg = _VPAD // _NS
    pltpu.sync_copy(lut_hbm.at[pl.ds(sid * seg, seg)],
                    lut_sp.at[pl.ds(sid * seg, seg)])
    plsc.subcore_barrier()
    rb = wid * rows_per_w

    def chunk_body(c, _):
        r0 = rb + c * _CH
        pltpu.sync_copy(idx_hbm.at[pl.ds(r0, _CH), :], idx_v)

        def grp_body(g, _):
            handles = [
                pltpu.async_copy(lut_sp.at[idx_v.at[g * _K + k]],
                                 out_v.at[g * _K + k], sem)
                for k in range(_K)
            ]
            for h in handles:
                h.wait()
            return 0

        lax.fori_loop(0, _CH // _K, grp_body, 0)
        pltpu.sync_copy(out_v, out_hbm.at[pl.ds(r0, _CH), :])
        return 0

    lax.fori_loop(0, rows_per_w // _CH, chunk_body, 0)


def _gather(lut, x):
    B, F = x.shape
    rows_per_w = B // _NW
    mesh = plsc.VectorSubcoreMesh(core_axis_name="c", subcore_axis_name="s")
    return pl.kernel(
        functools.partial(_gather_body, rows_per_w, F),
        out_type=jax.ShapeDtypeStruct((B, F), jnp.float32),
        mesh=mesh,
        scratch_types=[
            pltpu.VMEM_SHARED((_VPAD,), jnp.float32),
            pltpu.VMEM((_CH, F), jnp.int32),
            pltpu.VMEM((_CH, F), jnp.float32),
            pltpu.SemaphoreType.DMA,
        ],
    )(lut, x)


def kernel(x, table, W1, b1, W2, b2):
    lut = _build_lut(table, W1, b1, W2, b2)
    B, F = x.shape
    out2d = _gather(lut, x)
    return out2d.reshape(B, F, 1)


# speedup vs baseline: 1.0817x; 1.0024x over previous
import functools

import jax
import jax.numpy as jnp
from jax import lax
from jax.experimental import pallas as pl
from jax.experimental.pallas import tpu as pltpu
from jax.experimental.pallas import tpu_sc as plsc

VOCAB = 100000
EMB_DIM = 128
UNITS = 512


_VPAD = 100352
_VBLK = 7168
_GRID = _VPAD // _VBLK


def _lut_body(tab_ref, w1_ref, b1_ref, w2_ref, b2_ref, out_ref):
    h = jnp.maximum(tab_ref[...], 0.0).astype(jnp.bfloat16)
    a = jnp.dot(h, w1_ref[...], preferred_element_type=jnp.float32) + b1_ref[...]
    a = jnp.maximum(a, 0.0)
    o = jnp.dot(a, w2_ref[...], preferred_element_type=jnp.float32) + b2_ref[...]
    out_ref[...] = o.reshape(_VBLK // 128, 128)


def _build_lut(table, W1, b1, W2, b2):
    b1r = b1.reshape(1, UNITS)
    b2r = b2.reshape(1, 1)
    out = pl.pallas_call(
        _lut_body,
        grid=(_GRID,),
        in_specs=[
            pl.BlockSpec((_VBLK, EMB_DIM), lambda i: (i, 0)),
            pl.BlockSpec((EMB_DIM, UNITS), lambda i: (0, 0)),
            pl.BlockSpec((1, UNITS), lambda i: (0, 0)),
            pl.BlockSpec((UNITS, 1), lambda i: (0, 0)),
            pl.BlockSpec((1, 1), lambda i: (0, 0)),
        ],
        out_specs=pl.BlockSpec((_VBLK // 128, 128), lambda i: (i, 0)),
        out_shape=jax.ShapeDtypeStruct((_VPAD // 128, 128), jnp.float32),
    )(table, W1.astype(jnp.bfloat16), b1r, W2, b2r)
    return out.reshape(_VPAD)


_NC = 2
_NS = 16
_NW = _NC * _NS


_CH = 256
_K = 64


def _gather_body(rows_per_w, n_fields, lut_hbm, idx_hbm, out_hbm,
                 lut_sp, idx_v, out_v, sem):
    sid = lax.axis_index("s")
    wid = sid * _NC + lax.axis_index("c")
    seg = _VPAD // _NS
    pltpu.sync_copy(lut_hbm.at[pl.ds(sid * seg, seg)],
                    lut_sp.at[pl.ds(sid * seg, seg)])
    plsc.subcore_barrier()
    rb = wid * rows_per_w

    def chunk_body(c, _):
        r0 = rb + c * _CH
        pltpu.sync_copy(idx_hbm.at[pl.ds(r0, _CH), :], idx_v)

        def grp_body(g, _):
            handles = [
                pltpu.async_copy(lut_sp.at[idx_v.at[g * _K + k]],
                                 out_v.at[g * _K + k], sem)
                for k in range(_K)
            ]
            for h in handles:
                h.wait()
            return 0

        lax.fori_loop(0, _CH // _K, grp_body, 0)
        pltpu.sync_copy(out_v, out_hbm.at[pl.ds(r0, _CH), :])
        return 0

    lax.fori_loop(0, rows_per_w // _CH, chunk_body, 0)


def _gather(lut, x):
    B, F = x.shape
    rows_per_w = B // _NW
    mesh = plsc.VectorSubcoreMesh(core_axis_name="c", subcore_axis_name="s")
    return pl.kernel(
        functools.partial(_gather_body, rows_per_w, F),
        out_type=jax.ShapeDtypeStruct((B, F), jnp.float32),
        mesh=mesh,
        scratch_types=[
            pltpu.VMEM_SHARED((_VPAD,), jnp.float32),
            pltpu.VMEM((_CH, F), jnp.int32),
            pltpu.VMEM((_CH, F), jnp.float32),
            pltpu.SemaphoreType.DMA,
        ],
    )(lut, x)


def kernel(x, table, W1, b1, W2, b2):
    lut = _build_lut(table, W1, b1, W2, b2)
    B, F = x.shape
    out2d = _gather(lut, x)
    return out2d.reshape(B, F, 1)
